# baseline (speedup 1.0000x reference)
import jax
import jax.numpy as jnp
from jax import lax
from jax.experimental import pallas as pl
from jax.experimental.pallas import tpu as pltpu
from jax.experimental.pallas import tpu_sc as plsc

N = 10000
E = 320000
D = 128
H = 128

NC = 2
NS = 16
CHUNK = 128
GROUPS = CHUNK // 16
CPT = 160
EPAD = CPT * NS * CHUNK
ROWS_PT = 632
ROWS_LAST = N - 15 * ROWS_PT

_mesh = plsc.VectorSubcoreMesh(core_axis_name="c", subcore_axis_name="s")
_sc_params = pltpu.CompilerParams(needs_layout_passes=False)


def _xw_body(x_ref, w_ref, o_ref):
    o_ref[...] = jnp.dot(x_ref[...], w_ref[...], preferred_element_type=jnp.float32)


def _xw(x, w):
    return pl.pallas_call(
        _xw_body,
        grid=(10,),
        in_specs=[
            pl.BlockSpec((N // 10, D), lambda i: (i, 0)),
            pl.BlockSpec((D, H), lambda i: (0, 0)),
        ],
        out_specs=pl.BlockSpec((N // 10, H), lambda i: (i, 0)),
        out_shape=jax.ShapeDtypeStruct((N, H), jnp.float32),
    )(x, w)


def _deg_body(sd_hbm, w_hbm, zeros_hbm, out_hbm, meta_v, w_v, deg_sh,
              sem_s, sem_m):
    c = lax.axis_index("c")
    s = lax.axis_index("s")

    @pl.when((c == 0) & (s == 0))
    def _():
        pltpu.sync_copy(zeros_hbm, deg_sh)

    @pl.when(c == 0)
    def _():
        for b in range(2):
            pltpu.sync_copy(sd_hbm.at[s, b], meta_v.at[b])
            pltpu.sync_copy(w_hbm.at[s, b], w_v.at[b])

    plsc.subcore_barrier()

    @pl.when(c == 0)
    def _():
        def wait_scat(b):
            pltpu.make_async_copy(w_v.at[b, 0], deg_sh.at[meta_v.at[b, 1]],
                                  sem_s.at[b]).wait()

        def step(t, carry):
            for b in range(4):
                j = 4 * t + b
                b2 = (b + 2) % 4

                @pl.when(j >= 2)
                def _():
                    wait_scat(b2)

                @pl.when(j + 2 < CPT)
                def _():
                    pltpu.async_copy(sd_hbm.at[s, j + 2], meta_v.at[b2],
                                     sem_m.at[b2])
                    pltpu.async_copy(w_hbm.at[s, j + 2], w_v.at[b2],
                                     sem_m.at[b2])

                @pl.when(j >= 2)
                def _():
                    pltpu.make_async_copy(sd_hbm.at[s, j], meta_v.at[b],
                                          sem_m.at[b]).wait()
                    pltpu.make_async_copy(w_hbm.at[s, j], w_v.at[b],
                                          sem_m.at[b]).wait()

                pltpu.async_copy(w_v.at[b, 0], deg_sh.at[meta_v.at[b, 1]],
                                 sem_s.at[b], add=True)
            return carry

        lax.fori_loop(0, CPT // 4, step, 0)
        wait_scat(2)
        wait_scat(3)

    plsc.subcore_barrier()

    @pl.when((c == 0) & (s == 0))
    def _():
        pltpu.sync_copy(deg_sh, out_hbm)


_deg_kernel = pl.kernel(
    _deg_body,
    out_type=jax.ShapeDtypeStruct((N,), jnp.float32),
    mesh=_mesh,
    scratch_types=[
        pltpu.VMEM((4, 2, CHUNK), jnp.int32),
        pltpu.VMEM((4, 1, CHUNK), jnp.float32),
        pltpu.VMEM_SHARED((N,), jnp.float32),
        pltpu.SemaphoreType.DMA((4,)),
        pltpu.SemaphoreType.DMA((4,)),
    ],
    compiler_params=_sc_params,
)


def _dinv_body(p_ref, o_ref):
    o_ref[...] = lax.rsqrt(p_ref[...] + 1.0)[:, None]


def _dinv(deg):
    return pl.pallas_call(
        _dinv_body,
        out_shape=jax.ShapeDtypeStruct((N, 1), jnp.float32),
    )(deg)


_GATHER_DNUMS = lax.GatherDimensionNumbers(
    offset_dims=(), collapsed_slice_dims=(0,), start_index_map=(0,))


def _lane_bcast(v, e):
    idx = jnp.full((16, 1), e, jnp.int32)
    return lax.gather(v, idx, _GATHER_DNUMS, (1,),
                      mode=lax.GatherScatterMode.PROMISE_IN_BOUNDS)


def _msg_body(sd_hbm, w_hbm, dinv_hbm, xw_hbm, zeros_hbm, out_hbm,
              meta_v, w_v, dinv_v, rows_v, acc_sh, sem_g, sem_s, sem_m):
    c = lax.axis_index("c")
    s = lax.axis_index("s")

    def start_gather(j, m, b):
        pltpu.async_copy(xw_hbm.at[meta_v.at[m, 0]], rows_v.at[b],
                         sem_g.at[b])

    def wait_gather(m, b):
        pltpu.make_async_copy(xw_hbm.at[meta_v.at[m, 0]], rows_v.at[b],
                              sem_g.at[b]).wait()

    def start_scatter(m, b):
        pltpu.async_copy(rows_v.at[b], acc_sh.at[meta_v.at[m, 1]],
                         sem_s.at[b], add=True)

    def wait_scatter(m, b):
        pltpu.make_async_copy(rows_v.at[b], acc_sh.at[meta_v.at[m, 1]],
                              sem_s.at[b]).wait()

    @pl.when(c == 0)
    def _():
        @pl.when(s < NS - 1)
        def _():
            pltpu.sync_copy(zeros_hbm.at[pl.ds(s * ROWS_PT, ROWS_PT)],
                            acc_sh.at[pl.ds(s * ROWS_PT, ROWS_PT)])

        @pl.when(s == NS - 1)
        def _():
            pltpu.sync_copy(zeros_hbm.at[pl.ds((NS - 1) * ROWS_PT, ROWS_LAST)],
                            acc_sh.at[pl.ds((NS - 1) * ROWS_PT, ROWS_LAST)])

        pltpu.sync_copy(dinv_hbm, dinv_v)
        for sl in range(3):
            pltpu.sync_copy(sd_hbm.at[s, sl], meta_v.at[sl])
            pltpu.sync_copy(w_hbm.at[s, sl], w_v.at[sl])
        start_gather(0, 0, 0)

    plsc.subcore_barrier()

    @pl.when(c == 0)
    def _():
        def step(t, carry):
            for k in range(4):
                j = 4 * t + k
                b = k % 2
                b1 = (k + 1) % 2
                m = k
                m1 = (k + 1) % 4
                m3 = (k + 3) % 4

                @pl.when(j >= 1)
                def _():
                    wait_scatter(m3, b1)

                @pl.when(j + 3 < CPT)
                def _():
                    pltpu.async_copy(sd_hbm.at[s, j + 3], meta_v.at[m3],
                                     sem_m.at[m3])
                    pltpu.async_copy(w_hbm.at[s, j + 3], w_v.at[m3],
                                     sem_m.at[m3])

                @pl.when((j >= 2) & (j + 1 < CPT))
                def _():
                    pltpu.make_async_copy(sd_hbm.at[s, j + 1], meta_v.at[m1],
                                          sem_m.at[m1]).wait()
                    pltpu.make_async_copy(w_hbm.at[s, j + 1], w_v.at[m1],
                                          sem_m.at[m1]).wait()

                @pl.when(j + 1 < CPT)
                def _():
                    start_gather(j + 1, m1, b1)

                wait_gather(m, b)

                def scale_group(g, carry2):
                    base = g * 16
                    fs16 = pl.ds(base, 16)
                    src16 = meta_v[m, 0, fs16]
                    a16 = w_v[m, 0, fs16] * plsc.load_gather(dinv_v, [src16])
                    for e in range(16):
                        bc = _lane_bcast(a16, e)
                        for f in range(8):
                            fs = pl.ds(f * 16, 16)
                            rows_v[b, base + e, fs] = (
                                rows_v[b, base + e, fs] * bc)
                    return carry2

                lax.fori_loop(0, GROUPS, scale_group, 0)
                start_scatter(m, b)
            return carry

        lax.fori_loop(0, CPT // 4, step, 0)
        wait_scatter(3, 1)

    plsc.subcore_barrier()

    @pl.when(c == 0)
    def _():
        @pl.when(s < NS - 1)
        def _():
            pltpu.sync_copy(acc_sh.at[pl.ds(s * ROWS_PT, ROWS_PT)],
                            out_hbm.at[pl.ds(s * ROWS_PT, ROWS_PT)])

        @pl.when(s == NS - 1)
        def _():
            pltpu.sync_copy(acc_sh.at[pl.ds((NS - 1) * ROWS_PT, ROWS_LAST)],
                            out_hbm.at[pl.ds((NS - 1) * ROWS_PT, ROWS_LAST)])


_msg_kernel = pl.kernel(
    _msg_body,
    out_type=jax.ShapeDtypeStruct((N, H), jnp.float32),
    mesh=_mesh,
    scratch_types=[
        pltpu.VMEM((4, 2, CHUNK), jnp.int32),
        pltpu.VMEM((4, 1, CHUNK), jnp.float32),
        pltpu.VMEM((N,), jnp.float32),
        pltpu.VMEM((2, CHUNK, H), jnp.float32),
        pltpu.VMEM_SHARED((N, H), jnp.float32),
        pltpu.SemaphoreType.DMA((2,)),
        pltpu.SemaphoreType.DMA((2,)),
        pltpu.SemaphoreType.DMA((4,)),
    ],
    compiler_params=_sc_params,
)


def _gate_body(p_ref, x_ref, xw_ref, h_ref, dv_ref, bg_ref,
               wux, wug, wuh, bu, wrx, wrg, wrh, br, wcx, wcg, wch, bc_,
               o_ref):
    dv = dv_ref[...]
    pre = dv * (p_ref[...] + dv * xw_ref[...]) + bg_ref[...]
    g = jax.nn.sigmoid(pre)
    x = x_ref[...]
    h = h_ref[...]

    def dot(a, b):
        return jnp.dot(a, b[...], preferred_element_type=jnp.float32)

    u = jax.nn.sigmoid(dot(x, wux) + dot(g, wug) + dot(h, wuh) + bu[...])
    r = jax.nn.sigmoid(dot(x, wrx) + dot(g, wrg) + dot(h, wrh) + br[...])
    cc = jnp.tanh(dot(x, wcx) + dot(g, wcg) + dot(r * h, wch) + bc_[...])
    o_ref[...] = u * h + (1.0 - u) * cc


def _gates(partial, x, xw, h, dinv, b_gcn, wu3, bu, wr3, br, wc3, bc_):
    R = 1000
    row = lambda i: (i, 0)
    full = lambda i: (0, 0)
    wspec = pl.BlockSpec((D, H), full)
    bspec = pl.BlockSpec((1, H), full)
    return pl.pallas_call(
        _gate_body,
        grid=(N // R,),
        in_specs=[
            pl.BlockSpec((R, H), row),
            pl.BlockSpec((R, D), row),
            pl.BlockSpec((R, H), row),
            pl.BlockSpec((R, H), row),
            pl.BlockSpec((R, 1), row),
            bspec,
            wspec, wspec, wspec, bspec,
            wspec, wspec, wspec, bspec,
            wspec, wspec, wspec, bspec,
        ],
        out_specs=pl.BlockSpec((R, H), row),
        out_shape=jax.ShapeDtypeStruct((N, H), jnp.float32),
    )(partial, x, xw, h, dinv, b_gcn,
      wu3[0], wu3[1], wu3[2], bu, wr3[0], wr3[1], wr3[2], br,
      wc3[0], wc3[1], wc3[2], bc_)


def kernel(x, edge_index, edge_weight, h, W_gcn, b_gcn, W_u, b_u, W_r, b_r, W_c, b_c):
    pad = EPAD - E
    src = jnp.pad(edge_index[0], (0, pad)).reshape(NS, CPT, CHUNK)
    dst = jnp.pad(edge_index[1], (0, pad)).reshape(NS, CPT, CHUNK)
    sd = jnp.stack([src, dst], axis=2)
    w_r = jnp.pad(edge_weight, (0, pad)).reshape(NS, CPT, 1, CHUNK)

    zeros_n = jnp.zeros((N,), jnp.float32)
    zeros_nh = jnp.zeros((N, H), jnp.float32)

    xw = _xw(x, W_gcn)
    deg = _deg_kernel(sd, w_r, zeros_n)
    dinv = _dinv(deg)
    msg_partial = _msg_kernel(sd, w_r, dinv.reshape(N), xw, zeros_nh)

    wu3 = W_u.reshape(3, D, H)
    wr3 = W_r.reshape(3, D, H)
    wc3 = W_c.reshape(3, D, H)
    return _gates(msg_partial, x, xw, h, dinv, b_gcn.reshape(1, H),
                  wu3, b_u.reshape(1, H), wr3, b_r.reshape(1, H),
                  wc3, b_c.reshape(1, H))


# speedup vs baseline: 1.3203x; 1.3203x over previous
import jax
import jax.numpy as jnp
from jax import lax
from jax.experimental import pallas as pl
from jax.experimental.pallas import tpu as pltpu
from jax.experimental.pallas import tpu_sc as plsc

N = 10000
E = 320000
D = 128
H = 128

NC = 2
NS = 16
NW = NC * NS
CHUNK = 128
GROUPS = CHUNK // 16
CF_CORE = 0
CF = 152
CS = 8
CMAX = CF
EPT_PAIR = CHUNK * (CF + CS)
EPAD = EPT_PAIR * NS
ROWS_PT = 632
ROWS_LAST = N - 15 * ROWS_PT

_mesh = plsc.VectorSubcoreMesh(core_axis_name="c", subcore_axis_name="s")
_sc_params = pltpu.CompilerParams(needs_layout_passes=False)


def _xw_body(x_ref, w_ref, o_ref):
    o_ref[...] = jnp.dot(x_ref[...], w_ref[...], preferred_element_type=jnp.float32)


def _xw(x, w):
    return pl.pallas_call(
        _xw_body,
        grid=(10,),
        in_specs=[
            pl.BlockSpec((N // 10, D), lambda i: (i, 0)),
            pl.BlockSpec((D, H), lambda i: (0, 0)),
        ],
        out_specs=pl.BlockSpec((N // 10, H), lambda i: (i, 0)),
        out_shape=jax.ShapeDtypeStruct((N, H), jnp.float32),
    )(x, w)


def _deg_body(sd_hbm, w_hbm, zeros_hbm, out_hbm, meta_v, w_v, deg_sh,
              sem_s, sem_m):
    c = lax.axis_index("c")
    s = lax.axis_index("s")
    wid = s * NC + c
    nch = jnp.where(c == CF_CORE, CF, CS)

    @pl.when(s == 0)
    def _():
        pltpu.sync_copy(zeros_hbm, deg_sh)

    for b in range(2):
        pltpu.sync_copy(sd_hbm.at[wid, b], meta_v.at[b])
        pltpu.sync_copy(w_hbm.at[wid, b], w_v.at[b])
    plsc.subcore_barrier()

    def wait_scat(b):
        pltpu.make_async_copy(w_v.at[b, 0], deg_sh.at[meta_v.at[b, 1]],
                              sem_s.at[b]).wait()

    def step(t, carry):
        for b in range(4):
            j = 4 * t + b
            b2 = (b + 2) % 4

            @pl.when(j >= 2)
            def _():
                wait_scat(b2)

            @pl.when(j + 2 < nch)
            def _():
                pltpu.async_copy(sd_hbm.at[wid, j + 2], meta_v.at[b2],
                                 sem_m.at[b2])
                pltpu.async_copy(w_hbm.at[wid, j + 2], w_v.at[b2],
                                 sem_m.at[b2])

            @pl.when(j >= 2)
            def _():
                pltpu.make_async_copy(sd_hbm.at[wid, j], meta_v.at[b],
                                      sem_m.at[b]).wait()
                pltpu.make_async_copy(w_hbm.at[wid, j], w_v.at[b],
                                      sem_m.at[b]).wait()

            pltpu.async_copy(w_v.at[b, 0], deg_sh.at[meta_v.at[b, 1]],
                             sem_s.at[b], add=True)
        return carry

    lax.fori_loop(0, nch // 4, step, 0)
    wait_scat(2)
    wait_scat(3)
    plsc.subcore_barrier()

    @pl.when(s == 0)
    def _():
        pltpu.sync_copy(deg_sh, out_hbm.at[c])


_deg_kernel = pl.kernel(
    _deg_body,
    out_type=jax.ShapeDtypeStruct((NC, N), jnp.float32),
    mesh=_mesh,
    scratch_types=[
        pltpu.VMEM((4, 2, CHUNK), jnp.int32),
        pltpu.VMEM((4, 1, CHUNK), jnp.float32),
        pltpu.VMEM_SHARED((N,), jnp.float32),
        pltpu.SemaphoreType.DMA((4,)),
        pltpu.SemaphoreType.DMA((4,)),
    ],
    compiler_params=_sc_params,
)


def _dinv_body(p_ref, o_ref):
    o_ref[...] = lax.rsqrt(p_ref[0] + p_ref[1] + 1.0)[:, None]


def _dinv(partials):
    return pl.pallas_call(
        _dinv_body,
        out_shape=jax.ShapeDtypeStruct((N, 1), jnp.float32),
    )(partials)


_GATHER_DNUMS = lax.GatherDimensionNumbers(
    offset_dims=(), collapsed_slice_dims=(0,), start_index_map=(0,))


def _lane_bcast(v, e):
    idx = jnp.full((16, 1), e, jnp.int32)
    return lax.gather(v, idx, _GATHER_DNUMS, (1,),
                      mode=lax.GatherScatterMode.PROMISE_IN_BOUNDS)


def _msg_body(sd_hbm, w_hbm, dinv_hbm, xw_hbm, zeros_hbm, out_hbm,
              meta_v, w_v, dinv_v, rows_v, acc_sh, sem_g, sem_s, sem_m):
    c = lax.axis_index("c")
    s = lax.axis_index("s")
    wid = s * NC + c
    nch = jnp.where(c == CF_CORE, CF, CS)

    @pl.when(s < NS - 1)
    def _():
        pltpu.sync_copy(zeros_hbm.at[pl.ds(s * ROWS_PT, ROWS_PT)],
                        acc_sh.at[pl.ds(s * ROWS_PT, ROWS_PT)])

    @pl.when(s == NS - 1)
    def _():
        pltpu.sync_copy(zeros_hbm.at[pl.ds((NS - 1) * ROWS_PT, ROWS_LAST)],
                        acc_sh.at[pl.ds((NS - 1) * ROWS_PT, ROWS_LAST)])

    pltpu.sync_copy(dinv_hbm, dinv_v)
    for sl in range(3):
        pltpu.sync_copy(sd_hbm.at[wid, sl], meta_v.at[sl])
        pltpu.sync_copy(w_hbm.at[wid, sl], w_v.at[sl])

    def start_gather(j, m, b):
        pltpu.async_copy(xw_hbm.at[meta_v.at[m, 0]], rows_v.at[b],
                         sem_g.at[b])

    def wait_gather(m, b):
        pltpu.make_async_copy(xw_hbm.at[meta_v.at[m, 0]], rows_v.at[b],
                              sem_g.at[b]).wait()

    def start_scatter(m, b):
        pltpu.async_copy(rows_v.at[b], acc_sh.at[meta_v.at[m, 1]],
                         sem_s.at[b], add=True)

    def wait_scatter(m, b):
        pltpu.make_async_copy(rows_v.at[b], acc_sh.at[meta_v.at[m, 1]],
                              sem_s.at[b]).wait()

    start_gather(0, 0, 0)
    plsc.subcore_barrier()

    def step(t, carry):
        for k in range(4):
            j = 4 * t + k
            b = k % 2
            b1 = (k + 1) % 2
            m = k
            m1 = (k + 1) % 4
            m3 = (k + 3) % 4

            @pl.when(j >= 1)
            def _():
                wait_scatter(m3, b1)

            @pl.when(j + 3 < nch)
            def _():
                pltpu.async_copy(sd_hbm.at[wid, j + 3], meta_v.at[m3],
                                 sem_m.at[m3])
                pltpu.async_copy(w_hbm.at[wid, j + 3], w_v.at[m3],
                                 sem_m.at[m3])

            @pl.when((j >= 2) & (j + 1 < nch))
            def _():
                pltpu.make_async_copy(sd_hbm.at[wid, j + 1], meta_v.at[m1],
                                      sem_m.at[m1]).wait()
                pltpu.make_async_copy(w_hbm.at[wid, j + 1], w_v.at[m1],
                                      sem_m.at[m1]).wait()

            @pl.when(j + 1 < nch)
            def _():
                start_gather(j + 1, m1, b1)

            wait_gather(m, b)

            def scale_group(g, carry2):
                base = g * 16
                fs16 = pl.ds(base, 16)
                src16 = meta_v[m, 0, fs16]
                a16 = w_v[m, 0, fs16] * plsc.load_gather(dinv_v, [src16])
                for e in range(16):
                    bc = _lane_bcast(a16, e)
                    for f in range(8):
                        fs = pl.ds(f * 16, 16)
                        rows_v[b, base + e, fs] = rows_v[b, base + e, fs] * bc
                return carry2

            lax.fori_loop(0, GROUPS, scale_group, 0)
            start_scatter(m, b)
        return carry

    lax.fori_loop(0, nch // 4, step, 0)
    wait_scatter(3, 1)
    plsc.subcore_barrier()

    @pl.when(s < NS - 1)
    def _():
        pltpu.sync_copy(acc_sh.at[pl.ds(s * ROWS_PT, ROWS_PT)],
                        out_hbm.at[c, pl.ds(s * ROWS_PT, ROWS_PT)])

    @pl.when(s == NS - 1)
    def _():
        pltpu.sync_copy(acc_sh.at[pl.ds((NS - 1) * ROWS_PT, ROWS_LAST)],
                        out_hbm.at[c, pl.ds((NS - 1) * ROWS_PT, ROWS_LAST)])


_msg_kernel = pl.kernel(
    _msg_body,
    out_type=jax.ShapeDtypeStruct((NC, N, H), jnp.float32),
    mesh=_mesh,
    scratch_types=[
        pltpu.VMEM((4, 2, CHUNK), jnp.int32),
        pltpu.VMEM((4, 1, CHUNK), jnp.float32),
        pltpu.VMEM((N,), jnp.float32),
        pltpu.VMEM((2, CHUNK, H), jnp.float32),
        pltpu.VMEM_SHARED((N, H), jnp.float32),
        pltpu.SemaphoreType.DMA((2,)),
        pltpu.SemaphoreType.DMA((2,)),
        pltpu.SemaphoreType.DMA((4,)),
    ],
    compiler_params=_sc_params,
)


def _gate_body(p_ref, x_ref, xw_ref, h_ref, dv_ref, bg_ref,
               wux, wug, wuh, bu, wrx, wrg, wrh, br, wcx, wcg, wch, bc_,
               o_ref):
    dv = dv_ref[...]
    pre = dv * (p_ref[0] + p_ref[1] + dv * xw_ref[...]) + bg_ref[...]
    g = jax.nn.sigmoid(pre)
    x = x_ref[...]
    h = h_ref[...]

    def dot(a, b):
        return jnp.dot(a, b[...], preferred_element_type=jnp.float32)

    u = jax.nn.sigmoid(dot(x, wux) + dot(g, wug) + dot(h, wuh) + bu[...])
    r = jax.nn.sigmoid(dot(x, wrx) + dot(g, wrg) + dot(h, wrh) + br[...])
    cc = jnp.tanh(dot(x, wcx) + dot(g, wcg) + dot(r * h, wch) + bc_[...])
    o_ref[...] = u * h + (1.0 - u) * cc


def _gates(partials, x, xw, h, dinv, b_gcn, wu3, bu, wr3, br, wc3, bc_):
    R = 1000
    row = lambda i: (i, 0)
    full = lambda i: (0, 0)
    wspec = pl.BlockSpec((D, H), full)
    bspec = pl.BlockSpec((1, H), full)
    return pl.pallas_call(
        _gate_body,
        grid=(N // R,),
        in_specs=[
            pl.BlockSpec((NC, R, H), lambda i: (0, i, 0)),
            pl.BlockSpec((R, D), row),
            pl.BlockSpec((R, H), row),
            pl.BlockSpec((R, H), row),
            pl.BlockSpec((R, 1), row),
            bspec,
            wspec, wspec, wspec, bspec,
            wspec, wspec, wspec, bspec,
            wspec, wspec, wspec, bspec,
        ],
        out_specs=pl.BlockSpec((R, H), row),
        out_shape=jax.ShapeDtypeStruct((N, H), jnp.float32),
    )(partials, x, xw, h, dinv, b_gcn,
      wu3[0], wu3[1], wu3[2], bu, wr3[0], wr3[1], wr3[2], br,
      wc3[0], wc3[1], wc3[2], bc_)


def kernel(x, edge_index, edge_weight, h, W_gcn, b_gcn, W_u, b_u, W_r, b_r, W_c, b_c):
    pad = EPAD - E

    def split(arr):
        a = jnp.pad(arr, (0, pad))
        nf = NS * CF * CHUNK
        fast = a[:nf].reshape(NS, CF, CHUNK)
        slow = jnp.pad(a[nf:].reshape(NS, CS, CHUNK),
                       ((0, 0), (0, CF - CS), (0, 0)))
        pair = [fast, slow] if CF_CORE == 0 else [slow, fast]
        return jnp.stack(pair, axis=1).reshape(NW, CMAX, CHUNK)

    src = split(edge_index[0])
    dst = split(edge_index[1])
    sd = jnp.stack([src, dst], axis=2)
    w_r = split(edge_weight).reshape(NW, CMAX, 1, CHUNK)

    zeros_n = jnp.zeros((N,), jnp.float32)
    zeros_nh = jnp.zeros((N, H), jnp.float32)

    xw = _xw(x, W_gcn)
    deg_partials = _deg_kernel(sd, w_r, zeros_n)
    dinv = _dinv(deg_partials)
    msg_partials = _msg_kernel(sd, w_r, dinv.reshape(N), xw, zeros_nh)

    wu3 = W_u.reshape(3, D, H)
    wr3 = W_r.reshape(3, D, H)
    wc3 = W_c.reshape(3, D, H)
    return _gates(msg_partials, x, xw, h, dinv, b_gcn.reshape(1, H),
                  wu3, b_u.reshape(1, H), wr3, b_r.reshape(1, H),
                  wc3, b_c.reshape(1, H))
